# Initial kernel scaffold; baseline (speedup 1.0000x reference)
import jax
import jax.numpy as jnp
from jax.experimental import pallas as pl


def kernel(question_ids, type_ids, difficulty_ids, discriminate_ids, question_rel_embeds, Wq, Wt, Wd, Wdisc, Wpos, ln_gamma, ln_beta):
    raise NotImplementedError("write your pallas kernel here")


# speedup vs baseline: 2.1736x; 2.1736x over previous
import functools

import jax
import jax.numpy as jnp
from jax import lax
from jax.experimental import pallas as pl
from jax.experimental.pallas import tpu as pltpu
from jax.experimental.pallas import tpu_sc as plsc

_B, _L, _H = 1024, 200, 64
_N = _B * _L
_GW = 128
_BS = 16
_R = _BS * _L
_NT = 16
_EPS = 1e-12


def _sc_gather(table, idx_row):
    mesh = plsc.VectorSubcoreMesh(core_axis_name="core", subcore_axis_name="subcore")

    @functools.partial(
        pl.kernel,
        out_type=jax.ShapeDtypeStruct((_N, _H), table.dtype),
        mesh=mesh,
        compiler_params=pltpu.CompilerParams(use_tc_tiling_on_sc=False),
    )
    def gather_kernel(table_hbm, idx_hbm, out_hbm):
        def body(i_vmem, o_vmem):
            pltpu.sync_copy(table_hbm.at[i_vmem.at[0]], o_vmem)

        pltpu.emit_pipeline(
            body,
            grid=(_N // _GW,),
            in_specs=[pl.BlockSpec((1, _GW), index_map=lambda i: (0, i))],
            out_specs=[pl.BlockSpec((_GW, _H), index_map=lambda i: (i, 0))],
            core_axis_name=("core", "subcore"),
            dimension_semantics=(pltpu.PARALLEL,),
        )(idx_hbm, out_hbm)

    return gather_kernel(table, idx_row)


def _tc_body(q_ref, rel_ref, qid_ref, sid_ref, wsmall_ref, wpos_ref,
             gamma_ref, beta_ref, o_ref):
    rel = rel_ref[...].reshape(_R, _H)
    qid = qid_ref[...]
    sid = sid_ref[...]
    x = q_ref[...] + jnp.where(qid == 0, 0.0, rel)
    cols = lax.broadcasted_iota(jnp.int32, (_R, 3 * _NT), 1)
    t = sid >> 8
    d = (sid >> 4) & 15
    dc = sid & 15
    oh = (cols == t) | (cols == (d + _NT)) | (cols == (dc + 2 * _NT))
    small = jnp.dot(oh.astype(jnp.bfloat16), wsmall_ref[...],
                    preferred_element_type=jnp.float32)
    x = x + small + wpos_ref[...]
    mu = jnp.mean(x, axis=-1, keepdims=True)
    xc = x - mu
    var = jnp.mean(xc * xc, axis=-1, keepdims=True)
    y = xc * lax.rsqrt(var + _EPS)
    o_ref[...] = (y * gamma_ref[...] + beta_ref[...]).reshape(_BS, _L, _H)


def kernel(question_ids, type_ids, difficulty_ids, discriminate_ids,
           question_rel_embeds, Wq, Wt, Wd, Wdisc, Wpos, ln_gamma, ln_beta):
    seq_len = question_ids.shape[1]
    idx_dtype = jnp.int64 if question_ids.dtype == jnp.int64 else jnp.int32
    position_ids = jnp.arange(seq_len, dtype=idx_dtype)[None, :]

    qid32 = question_ids.astype(jnp.int32)
    qrows = _sc_gather(Wq, qid32.reshape(1, _N))

    qid_col = qid32.reshape(_N, 1)
    sid_col = ((type_ids.astype(jnp.int32) << 8)
               | (difficulty_ids.astype(jnp.int32) << 4)
               | discriminate_ids.astype(jnp.int32)).reshape(_N, 1)
    wsmall = jnp.concatenate([
        Wt,
        jnp.pad(Wd, ((0, _NT - Wd.shape[0]), (0, 0))),
        jnp.pad(Wdisc, ((0, _NT - Wdisc.shape[0]), (0, 0))),
    ], axis=0).astype(jnp.bfloat16)
    wpos_t = jnp.tile(Wpos[:seq_len], (_BS, 1))
    gamma2 = ln_gamma.reshape(1, _H)
    beta2 = ln_beta.reshape(1, _H)

    x = pl.pallas_call(
        _tc_body,
        grid=(_B // _BS,),
        in_specs=[
            pl.BlockSpec((_R, _H), lambda i: (i, 0)),
            pl.BlockSpec((_BS, _L, _H), lambda i: (i, 0, 0)),
            pl.BlockSpec((_R, 1), lambda i: (i, 0)),
            pl.BlockSpec((_R, 1), lambda i: (i, 0)),
            pl.BlockSpec((3 * _NT, _H), lambda i: (0, 0)),
            pl.BlockSpec((_R, _H), lambda i: (0, 0)),
            pl.BlockSpec((1, _H), lambda i: (0, 0)),
            pl.BlockSpec((1, _H), lambda i: (0, 0)),
        ],
        out_specs=pl.BlockSpec((_BS, _L, _H), lambda i: (i, 0, 0)),
        out_shape=jax.ShapeDtypeStruct((_B, _L, _H), jnp.float32),
    )(qrows, question_rel_embeds, qid_col, sid_col, wsmall, wpos_t, gamma2, beta2)

    return (x, position_ids)


# speedup vs baseline: 2.5032x; 1.1516x over previous
import functools

import jax
import jax.numpy as jnp
import numpy as np
from jax import lax
from jax.experimental import pallas as pl
from jax.experimental.pallas import tpu as pltpu
from jax.experimental.pallas import tpu_sc as plsc

_B, _L, _H = 1024, 200, 64
_N = _B * _L
_GW = 128
_BS = 16
_R = _BS * _L
_NT = 16
_EPS = 1e-12
_NW = 32
_PW = _N // _NW
_NWIN = _PW // _GW

_KM = np.zeros((2, _H), dtype=np.int32)
for _j in range(_H):
    if _j < 16:
        _KM[0, _j], _KM[1, _j] = _j << 9, 0xF << 9
    elif _j < 32:
        _KM[0, _j], _KM[1, _j] = (_j - 16) << 5, 0xF << 5
    elif _j < 48:
        _KM[0, _j], _KM[1, _j] = (_j - 32) << 1, 0xF << 1
    elif _j == 48:
        _KM[0, _j], _KM[1, _j] = 1, 1
    else:
        _KM[0, _j], _KM[1, _j] = 1 << 30, -1


def _sc_gather(table, idx_flat):
    mesh = plsc.VectorSubcoreMesh(core_axis_name="core", subcore_axis_name="subcore")

    @functools.partial(
        pl.kernel,
        out_type=jax.ShapeDtypeStruct((_N, 128), jnp.float32),
        mesh=mesh,
        compiler_params=pltpu.CompilerParams(use_tc_tiling_on_sc=False),
        scratch_types=[
            pltpu.VMEM((_PW,), jnp.int32),
            pltpu.VMEM((_GW, _H), jnp.float32),
            pltpu.VMEM((_GW, _H), jnp.float32),
            pltpu.SemaphoreType.DMA,
            pltpu.SemaphoreType.DMA,
            pltpu.SemaphoreType.DMA,
            pltpu.SemaphoreType.DMA,
        ],
    )
    def gather_kernel(table_hbm, idx_hbm, out_hbm, ibuf, rbuf0, rbuf1,
                      gsem0, gsem1, wsem0, wsem1):
        wid = lax.axis_index("subcore") * 2 + lax.axis_index("core")
        base = wid * _PW
        pltpu.sync_copy(idx_hbm.at[pl.ds(base, _PW)], ibuf)
        rbufs = (rbuf0, rbuf1)
        gsems = (gsem0, gsem1)
        wsems = (wsem0, wsem1)

        @pl.loop(0, _NWIN, step=2)
        def _(g):
            for k in (0, 1):
                gg = g + k
                r0 = base + gg * _GW

                @pl.when(gg >= 2)
                def _():
                    pltpu.make_async_copy(
                        rbufs[k],
                        out_hbm.at[pl.ds(r0 - 2 * _GW, _GW), pl.ds(0, _H)],
                        wsems[k],
                    ).wait()

                gather = pltpu.async_copy(
                    table_hbm.at[ibuf.at[pl.ds(gg * _GW, _GW)]],
                    rbufs[k], gsems[k])
                gather.wait()
                pltpu.async_copy(
                    rbufs[k],
                    out_hbm.at[pl.ds(r0, _GW), pl.ds(0, _H)],
                    wsems[k])

        for k in (0, 1):
            gg = _NWIN - 2 + k
            pltpu.make_async_copy(
                rbufs[k],
                out_hbm.at[pl.ds(base + gg * _GW, _GW), pl.ds(0, _H)],
                wsems[k],
            ).wait()

    return gather_kernel(table, idx_flat)


def _tc_body(q_ref, rel_ref, pid_ref, wsm_ref, wpos_ref, km_ref,
             gamma_ref, beta_ref, o_ref):
    rel = rel_ref[...].reshape(_R, _H)
    pid = pid_ref[...]
    kk = km_ref[0:1, :]
    mm = km_ref[1:2, :]
    oh = ((pid ^ kk) & mm) == 0
    smallm = jnp.dot(oh.astype(jnp.bfloat16), wsm_ref[...],
                     preferred_element_type=jnp.float32)
    small = smallm[:, :_H]
    m = smallm[:, _H:]
    x = q_ref[...][:, :_H] + small + wpos_ref[...] + rel * (1.0 - m)
    s1 = jnp.sum(x, axis=-1, keepdims=True)
    s2 = jnp.sum(x * x, axis=-1, keepdims=True)
    mu = s1 * (1.0 / _H)
    var = s2 * (1.0 / _H) - mu * mu
    y = (x - mu) * lax.rsqrt(var + _EPS)
    o_ref[...] = (y * gamma_ref[...] + beta_ref[...]).reshape(_BS, _L, _H)


def kernel(question_ids, type_ids, difficulty_ids, discriminate_ids,
           question_rel_embeds, Wq, Wt, Wd, Wdisc, Wpos, ln_gamma, ln_beta):
    seq_len = question_ids.shape[1]
    idx_dtype = jnp.int64 if question_ids.dtype == jnp.int64 else jnp.int32
    position_ids = jnp.arange(seq_len, dtype=idx_dtype)[None, :]

    qid32 = question_ids.astype(jnp.int32)
    qrows = _sc_gather(Wq, qid32.reshape(_N))

    pid = ((type_ids.astype(jnp.int32) << 9)
           | (difficulty_ids.astype(jnp.int32) << 5)
           | (discriminate_ids.astype(jnp.int32) << 1)
           | (qid32 == 0).astype(jnp.int32)).reshape(_N, 1)

    wsm = jnp.zeros((_H, 2 * _H), dtype=jnp.bfloat16)
    wsm = wsm.at[0:_NT, :_H].set(Wt.astype(jnp.bfloat16))
    wsm = wsm.at[_NT:_NT + Wd.shape[0], :_H].set(Wd.astype(jnp.bfloat16))
    wsm = wsm.at[2 * _NT:2 * _NT + Wdisc.shape[0], :_H].set(Wdisc.astype(jnp.bfloat16))
    wsm = wsm.at[48, _H:].set(jnp.ones((_H,), jnp.bfloat16))

    wpos_t = jnp.tile(Wpos[:seq_len], (_BS, 1))
    km = jnp.asarray(_KM)
    gamma2 = ln_gamma.reshape(1, _H)
    beta2 = ln_beta.reshape(1, _H)

    x = pl.pallas_call(
        _tc_body,
        grid=(_B // _BS,),
        in_specs=[
            pl.BlockSpec((_R, 128), lambda i: (i, 0)),
            pl.BlockSpec((_BS, _L, _H), lambda i: (i, 0, 0)),
            pl.BlockSpec((_R, 1), lambda i: (i, 0)),
            pl.BlockSpec((_H, 2 * _H), lambda i: (0, 0)),
            pl.BlockSpec((_R, _H), lambda i: (0, 0)),
            pl.BlockSpec((2, _H), lambda i: (0, 0)),
            pl.BlockSpec((1, _H), lambda i: (0, 0)),
            pl.BlockSpec((1, _H), lambda i: (0, 0)),
        ],
        out_specs=pl.BlockSpec((_BS, _L, _H), lambda i: (i, 0, 0)),
        out_shape=jax.ShapeDtypeStruct((_B, _L, _H), jnp.float32),
    )(qrows, question_rel_embeds, pid, wsm, wpos_t, km, gamma2, beta2)

    return (x, position_ids)
